# baseline (speedup 1.0000x reference)
import functools

import jax
import jax.numpy as jnp
from jax import lax
from jax.experimental import pallas as pl
from jax.experimental.pallas import tpu as pltpu
from jax.experimental.pallas import tpu_sc as plsc

_LANES = 128
_NUM_CLUSTERS = 100
_TC_BLK = 2048


def _sc_gather(user_indices, item_indices, emb_user, emb_item, user_reprs):
    B = user_indices.shape[0]
    D = emb_user.shape[1]
    NW = 32
    b_per_w = B // NW
    mesh = plsc.VectorSubcoreMesh(core_axis_name="c", subcore_axis_name="s")

    @functools.partial(
        pl.kernel,
        mesh=mesh,
        compiler_params=pltpu.CompilerParams(use_tc_tiling_on_sc=False),
        out_type=[
            jax.ShapeDtypeStruct((B, D), jnp.float32),
            jax.ShapeDtypeStruct((B, D), jnp.float32),
            jax.ShapeDtypeStruct((B, D), jnp.float32),
        ],
        scratch_types=[
            pltpu.VMEM((b_per_w,), jnp.int32),
            pltpu.VMEM((b_per_w,), jnp.int32),
            pltpu.VMEM((b_per_w, D), jnp.float32),
            pltpu.VMEM((b_per_w, D), jnp.float32),
            pltpu.VMEM((b_per_w, D), jnp.float32),
            pltpu.SemaphoreType.DMA,
            pltpu.SemaphoreType.DMA,
            pltpu.SemaphoreType.DMA,
        ],
    )
    def k(uidx_hbm, iidx_hbm, emb_u_hbm, emb_i_hbm, reprs_hbm,
          u_out, r_out, it_out,
          uidx_v, iidx_v, u_v, r_v, it_v, s1, s2, s3):
        wid = lax.axis_index("s") * 2 + lax.axis_index("c")
        base = wid * b_per_w
        pltpu.sync_copy(uidx_hbm.at[pl.ds(base, b_per_w)], uidx_v)
        pltpu.sync_copy(iidx_hbm.at[pl.ds(base, b_per_w)], iidx_v)
        c1 = pltpu.async_copy(emb_u_hbm.at[uidx_v], u_v, s1)
        c2 = pltpu.async_copy(reprs_hbm.at[uidx_v], r_v, s2)
        c3 = pltpu.async_copy(emb_i_hbm.at[iidx_v], it_v, s3)
        c1.wait()
        c2.wait()
        c3.wait()
        pltpu.sync_copy(u_v, u_out.at[pl.ds(base, b_per_w)])
        pltpu.sync_copy(r_v, r_out.at[pl.ds(base, b_per_w)])
        pltpu.sync_copy(it_v, it_out.at[pl.ds(base, b_per_w)])

    return k(user_indices, item_indices, emb_user, emb_item, user_reprs)


def _tc_body(u_ref, r_ref, it_ref, cent_t_ref, cent_ref, w_ref, b_ref, out_ref):
    r = r_ref[...]
    cent_t = cent_t_ref[...]
    scores = jnp.dot(r, cent_t, preferred_element_type=jnp.float32)
    cnorm = jnp.sum(cent_t * cent_t, axis=0, keepdims=True)
    scores = scores - 0.5 * cnorm
    cid = lax.broadcasted_iota(jnp.int32, scores.shape, 1)
    scores = jnp.where(cid < _NUM_CLUSTERS, scores, -jnp.inf)
    m = jnp.max(scores, axis=1, keepdims=True)
    nearest = jnp.min(jnp.where(scores == m, cid, _LANES), axis=1,
                      keepdims=True)
    onehot = (cid == nearest).astype(jnp.float32)
    proto = jnp.dot(onehot, cent_ref[...],
                    preferred_element_type=jnp.float32)
    prod = u_ref[...] * proto * it_ref[...]
    logits = jnp.sum(prod * w_ref[...], axis=1, keepdims=True) + b_ref[...]
    out_ref[...] = jax.nn.sigmoid(logits)


def _tc_forward(u, r, it, cluster_centers, W, b):
    B, D = u.shape
    C = cluster_centers.shape[0]
    cent = jnp.zeros((_LANES, D), jnp.float32).at[:C].set(cluster_centers)
    cent_t = cent.T
    w_row = W.reshape(1, D)
    b_11 = b.reshape(1, 1)
    blk = min(_TC_BLK, B)
    grid = (B // blk,)
    return pl.pallas_call(
        _tc_body,
        grid=grid,
        in_specs=[
            pl.BlockSpec((blk, D), lambda g: (g, 0)),
            pl.BlockSpec((blk, D), lambda g: (g, 0)),
            pl.BlockSpec((blk, D), lambda g: (g, 0)),
            pl.BlockSpec((D, _LANES), lambda g: (0, 0)),
            pl.BlockSpec((_LANES, D), lambda g: (0, 0)),
            pl.BlockSpec((1, D), lambda g: (0, 0)),
            pl.BlockSpec((1, 1), lambda g: (0, 0)),
        ],
        out_specs=pl.BlockSpec((blk, 1), lambda g: (g, 0)),
        out_shape=jax.ShapeDtypeStruct((B, 1), jnp.float32),
    )(u, r, it, cent_t, cent, w_row, b_11)


def kernel(user_indices, item_indices, emb_user, emb_item, user_reprs,
           cluster_centers, W, b):
    u, r, it = _sc_gather(user_indices, item_indices, emb_user, emb_item,
                          user_reprs)
    return _tc_forward(u, r, it, cluster_centers, W, b)
